# baseline (speedup 1.0000x reference)
import functools

import jax
import jax.numpy as jnp
from jax import lax
from jax.experimental import pallas as pl
from jax.experimental.pallas import tpu as pltpu
from jax.experimental.pallas import tpu_sc as plsc

PERIODS = (24, 168, 720, 8760)
D = 1024
ED = 256
NC = 2
NS = 16
NW = NC * NS
TOK = 4 * 4096
TPW = TOK // NW
T = 16
NCHUNK = TPW // T


def _body(x_hbm, t_hbm, e0, e1, e2, e3, out_hbm,
          xbuf, rows, e01, tloc, midx, xsem, gsem, osem):
    wid = lax.axis_index("s") * NC + lax.axis_index("c")
    base = wid * TPW

    pltpu.sync_copy(e0, e01.at[pl.ds(0, PERIODS[0] * ED)])
    pltpu.sync_copy(e1, e01.at[pl.ds(PERIODS[0] * ED, PERIODS[1] * ED)])
    pltpu.sync_copy(t_hbm.at[pl.ds(base, TPW)], tloc.at[pl.ds(0, TPW)])

    def mods(j, carry):
        tv = tloc[pl.ds(j * 16, 16)]
        midx[0, pl.ds(j * 16, 16)] = lax.rem(tv, PERIODS[2])
        return carry

    lax.fori_loop(0, TPW // 16, mods, 0)

    def start_g(c, b):
        pltpu.async_copy(
            e2.at[midx.at[0, pl.ds(c * T, T)]], rows.at[b, 0], gsem.at[b])
        pltpu.async_copy(
            e3.at[tloc.at[pl.ds(c * T, T)]], rows.at[b, 1], gsem.at[b])

    def start_x(c, b):
        tb = base + c * T
        pltpu.async_copy(x_hbm.at[pl.ds(tb, T)], xbuf.at[b], xsem.at[b])

    def wait_x(c, b):
        tb = base + c * T
        pltpu.make_async_copy(
            x_hbm.at[pl.ds(tb, T)], xbuf.at[b], xsem.at[b]).wait()

    def wait_g(c, b):
        pltpu.make_async_copy(
            e2.at[midx.at[0, pl.ds(c * T, T)]], rows.at[b, 0],
            gsem.at[b]).wait()
        pltpu.make_async_copy(
            e3.at[tloc.at[pl.ds(c * T, T)]], rows.at[b, 1],
            gsem.at[b]).wait()

    def start_out(c, b):
        tb = base + c * T
        pltpu.async_copy(xbuf.at[b], out_hbm.at[pl.ds(tb, T)], osem.at[b])

    def wait_out(c, b):
        tb = base + c * T
        pltpu.make_async_copy(
            xbuf.at[b], out_hbm.at[pl.ds(tb, T)], osem.at[b]).wait()

    def adds_e01(c, b):
        off = c * T

        @plsc.parallel_loop(0, T)
        def addtok(i):
            t = tloc[pl.ds(off + i, 16)][0]
            r0 = lax.rem(t, PERIODS[0]) * ED
            r1 = (lax.rem(t, PERIODS[1]) + PERIODS[0]) * ED
            for j in range(16):
                plsc.addupdate(xbuf.at[b, i, pl.ds(j * 16, 16)],
                               e01[pl.ds(r0 + j * 16, 16)])
            for j in range(16):
                plsc.addupdate(xbuf.at[b, i, pl.ds(ED + j * 16, 16)],
                               e01[pl.ds(r1 + j * 16, 16)])

    def adds_rows(c, b):
        @plsc.parallel_loop(0, T)
        def addtok(i):
            for j in range(16):
                plsc.addupdate(xbuf.at[b, i, pl.ds(2 * ED + j * 16, 16)],
                               rows[b, 0, i, pl.ds(j * 16, 16)])
            for j in range(16):
                plsc.addupdate(xbuf.at[b, i, pl.ds(3 * ED + j * 16, 16)],
                               rows[b, 1, i, pl.ds(j * 16, 16)])

    def adds(c, b):
        wait_x(c, b)
        adds_e01(c, b)
        wait_g(c, b)
        adds_rows(c, b)

    start_x(0, 0)
    start_g(0, 0)
    start_x(1, 1)
    start_g(1, 1)

    def step(c, b, pb, prefetch, first=False):
        adds(c, b)
        start_out(c, b)
        if prefetch:
            def pf():
                if not first:
                    wait_out(c - 1, pb)
                start_g(c + 2, pb)
                start_x(c + 2, pb)
            if first:
                pf()
            else:
                pl.when(c + 2 < NCHUNK)(pf)

    step(0, 0, 2, True, first=True)

    def triple(q, carry):
        for k in (1, 2, 3):
            c = 3 * q + k
            step(c, k % 3, (k + 2) % 3, True)
        return carry

    lax.fori_loop(0, (NCHUNK - 2) // 3, triple, 0)

    cL = NCHUNK - 1
    step(cL, cL % 3, 0, False)
    wait_out(cL - 2, (cL - 2) % 3)
    wait_out(cL - 1, (cL - 1) % 3)
    wait_out(cL, cL % 3)


@jax.jit
def _run(x2d, t1d, E0, E1, E2, E3):
    mesh = plsc.VectorSubcoreMesh(core_axis_name="c", subcore_axis_name="s")
    launch = functools.partial(
        pl.kernel,
        out_type=jax.ShapeDtypeStruct((TOK, D), jnp.float32),
        mesh=mesh,
        scratch_types=[
            pltpu.VMEM((3, T, D), jnp.float32),
            pltpu.VMEM((3, 2, T, ED), jnp.float32),
            pltpu.VMEM(((PERIODS[0] + PERIODS[1]) * ED,), jnp.float32),
            pltpu.VMEM((TPW + 16,), jnp.int32),
            pltpu.VMEM((2, TPW), jnp.int32),
            pltpu.SemaphoreType.DMA((3,)),
            pltpu.SemaphoreType.DMA((3,)),
            pltpu.SemaphoreType.DMA((3,)),
        ],
    )(_body)
    return launch(x2d, t1d, E0.reshape(-1), E1.reshape(-1), E2, E3)


def kernel(x, time_indices, E0, E1, E2, E3):
    B, S, _ = x.shape
    out = _run(
        x.reshape(TOK, D),
        time_indices.reshape(TOK).astype(jnp.int32),
        E0, E1, E2, E3,
    )
    return out.reshape(B, S, D)


# speedup vs baseline: 1.0492x; 1.0492x over previous
import functools

import jax
import jax.numpy as jnp
from jax import lax
from jax.experimental import pallas as pl
from jax.experimental.pallas import tpu as pltpu
from jax.experimental.pallas import tpu_sc as plsc

PERIODS = (24, 168, 720, 8760)
D = 1024
ED = 256
NC = 2
NS = 16
NW = NC * NS
TOK = 4 * 4096
TPW = TOK // NW
T = 16
NCHUNK = TPW // T


def _body(x_hbm, t_hbm, e0, e1, e2, e3, out_hbm,
          xbuf, rows, e01, tloc, midx, xsem, gsem, osem):
    wid = lax.axis_index("s") * NC + lax.axis_index("c")
    base = wid * TPW

    pltpu.sync_copy(e0, e01.at[pl.ds(0, PERIODS[0] * ED)])
    pltpu.sync_copy(e1, e01.at[pl.ds(PERIODS[0] * ED, PERIODS[1] * ED)])
    pltpu.sync_copy(t_hbm.at[pl.ds(base, TPW)], tloc.at[pl.ds(0, TPW)])

    def mods(j, carry):
        tv = tloc[pl.ds(j * 16, 16)]
        midx[0, pl.ds(j * 16, 16)] = lax.rem(tv, PERIODS[2])
        return carry

    lax.fori_loop(0, TPW // 16, mods, 0)

    def start_g(c, b):
        pltpu.async_copy(
            e2.at[midx.at[0, pl.ds(c * T, T)]], rows.at[b, 0], gsem.at[b])
        pltpu.async_copy(
            e3.at[tloc.at[pl.ds(c * T, T)]], rows.at[b, 1], gsem.at[b])

    def start_x(c, b):
        tb = base + c * T
        pltpu.async_copy(x_hbm.at[pl.ds(tb, T)], xbuf.at[b], xsem.at[b])

    def wait_x(c, b):
        tb = base + c * T
        pltpu.make_async_copy(
            x_hbm.at[pl.ds(tb, T)], xbuf.at[b], xsem.at[b]).wait()

    def wait_g(c, b):
        pltpu.make_async_copy(
            e2.at[midx.at[0, pl.ds(c * T, T)]], rows.at[b, 0],
            gsem.at[b]).wait()
        pltpu.make_async_copy(
            e3.at[tloc.at[pl.ds(c * T, T)]], rows.at[b, 1],
            gsem.at[b]).wait()

    def start_out(c, b):
        tb = base + c * T
        pltpu.async_copy(xbuf.at[b], out_hbm.at[pl.ds(tb, T)], osem.at[b])

    def wait_out(c, b):
        tb = base + c * T
        pltpu.make_async_copy(
            xbuf.at[b], out_hbm.at[pl.ds(tb, T)], osem.at[b]).wait()

    def adds_e01(c, b):
        off = c * T

        @plsc.parallel_loop(0, T)
        def addtok(i):
            t = tloc[pl.ds(off + i, 16)][0]
            r0 = lax.rem(t, PERIODS[0]) * ED
            r1 = (lax.rem(t, PERIODS[1]) + PERIODS[0]) * ED
            for j in range(16):
                plsc.addupdate(xbuf.at[b, i, pl.ds(j * 16, 16)],
                               e01[pl.ds(r0 + j * 16, 16)])
            for j in range(16):
                plsc.addupdate(xbuf.at[b, i, pl.ds(ED + j * 16, 16)],
                               e01[pl.ds(r1 + j * 16, 16)])

    def adds_rows(c, b):
        @plsc.parallel_loop(0, 2 * T)
        def addhalf(h):
            i = h // 2
            k = h % 2
            for j in range(16):
                plsc.addupdate(
                    xbuf.at[b, i, pl.ds((2 + k) * ED + j * 16, 16)],
                    rows[b, k, i, pl.ds(j * 16, 16)])

    def adds(c, b):
        wait_x(c, b)
        adds_e01(c, b)
        wait_g(c, b)
        adds_rows(c, b)

    start_x(0, 0)
    start_g(0, 0)
    start_x(1, 1)
    start_g(1, 1)

    def step(c, b, pb, prefetch, first=False):
        adds(c, b)
        start_out(c, b)
        if prefetch:
            def pf():
                if not first:
                    wait_out(c - 1, pb)
                start_g(c + 2, pb)
                start_x(c + 2, pb)
            if first:
                pf()
            else:
                pl.when(c + 2 < NCHUNK)(pf)

    step(0, 0, 2, True, first=True)

    def triple(q, carry):
        for k in (1, 2, 3):
            c = 3 * q + k
            step(c, k % 3, (k + 2) % 3, True)
        return carry

    lax.fori_loop(0, (NCHUNK - 2) // 3, triple, 0)

    cL = NCHUNK - 1
    step(cL, cL % 3, 0, False)
    wait_out(cL - 2, (cL - 2) % 3)
    wait_out(cL - 1, (cL - 1) % 3)
    wait_out(cL, cL % 3)


@jax.jit
def _run(x2d, t1d, E0, E1, E2, E3):
    mesh = plsc.VectorSubcoreMesh(core_axis_name="c", subcore_axis_name="s")
    launch = functools.partial(
        pl.kernel,
        out_type=jax.ShapeDtypeStruct((TOK, D), jnp.float32),
        mesh=mesh,
        scratch_types=[
            pltpu.VMEM((3, T, D), jnp.float32),
            pltpu.VMEM((3, 2, T, ED), jnp.float32),
            pltpu.VMEM(((PERIODS[0] + PERIODS[1]) * ED,), jnp.float32),
            pltpu.VMEM((TPW + 16,), jnp.int32),
            pltpu.VMEM((2, TPW), jnp.int32),
            pltpu.SemaphoreType.DMA((3,)),
            pltpu.SemaphoreType.DMA((3,)),
            pltpu.SemaphoreType.DMA((3,)),
        ],
    )(_body)
    return launch(x2d, t1d, E0.reshape(-1), E1.reshape(-1), E2, E3)


def kernel(x, time_indices, E0, E1, E2, E3):
    B, S, _ = x.shape
    out = _run(
        x.reshape(TOK, D),
        time_indices.reshape(TOK).astype(jnp.int32),
        E0, E1, E2, E3,
    )
    return out.reshape(B, S, D)


# speedup vs baseline: 1.0953x; 1.0440x over previous
import functools

import jax
import jax.numpy as jnp
from jax import lax
from jax.experimental import pallas as pl
from jax.experimental.pallas import tpu as pltpu
from jax.experimental.pallas import tpu_sc as plsc

PERIODS = (24, 168, 720, 8760)
D = 1024
ED = 256
NC = 2
NS = 16
NW = NC * NS
TOK = 4 * 4096
TPW = TOK // NW
T = 16
NCHUNK = TPW // T


def _body(x_hbm, t_hbm, e0, e1, e2, e3, out_hbm,
          xbuf, rows, e01, tloc, midx, xsem, gsem, osem):
    wid = lax.axis_index("s") * NC + lax.axis_index("c")
    base = wid * TPW

    pltpu.sync_copy(e0, e01.at[pl.ds(0, PERIODS[0] * ED)])
    pltpu.sync_copy(e1, e01.at[pl.ds(PERIODS[0] * ED, PERIODS[1] * ED)])
    pltpu.sync_copy(t_hbm.at[pl.ds(base, TPW)], tloc.at[pl.ds(0, TPW)])

    def mods(j, carry):
        tv = tloc[pl.ds(j * 16, 16)]
        midx[0, pl.ds(j * 16, 16)] = lax.rem(tv, PERIODS[2])
        return carry

    lax.fori_loop(0, TPW // 16, mods, 0)

    def start_g(c, b):
        pltpu.async_copy(
            e2.at[midx.at[0, pl.ds(c * T, T)]], rows.at[b, 0], gsem.at[b])
        pltpu.async_copy(
            e3.at[tloc.at[pl.ds(c * T, T)]], rows.at[b, 1], gsem.at[b])

    def start_x(c, b):
        tb = base + c * T
        pltpu.async_copy(x_hbm.at[pl.ds(tb, T)], xbuf.at[b], xsem.at[b])

    def wait_x(c, b):
        tb = base + c * T
        pltpu.make_async_copy(
            x_hbm.at[pl.ds(tb, T)], xbuf.at[b], xsem.at[b]).wait()

    def wait_g(c, b):
        pltpu.make_async_copy(
            e2.at[midx.at[0, pl.ds(c * T, T)]], rows.at[b, 0],
            gsem.at[b]).wait()
        pltpu.make_async_copy(
            e3.at[tloc.at[pl.ds(c * T, T)]], rows.at[b, 1],
            gsem.at[b]).wait()

    def start_out(c, b):
        tb = base + c * T
        pltpu.async_copy(xbuf.at[b], out_hbm.at[pl.ds(tb, T)], osem.at[b])

    def wait_out(c, b):
        tb = base + c * T
        pltpu.make_async_copy(
            xbuf.at[b], out_hbm.at[pl.ds(tb, T)], osem.at[b]).wait()

    def adds_e01(c, b):
        off = c * T

        @plsc.parallel_loop(0, 2 * T)
        def addhalf(h):
            i = h // 2
            k = h % 2
            t = tloc[pl.ds(off + i, 16)][0]
            p = PERIODS[0] + k * (PERIODS[1] - PERIODS[0])
            r = (lax.rem(t, p) + k * PERIODS[0]) * ED
            for j in range(16):
                plsc.addupdate(xbuf.at[b, i, pl.ds(k * ED + j * 16, 16)],
                               e01[pl.ds(r + j * 16, 16)])

    def adds_rows(c, b):
        @plsc.parallel_loop(0, 2 * T)
        def addhalf(h):
            i = h // 2
            k = h % 2
            for j in range(16):
                plsc.addupdate(
                    xbuf.at[b, i, pl.ds((2 + k) * ED + j * 16, 16)],
                    rows[b, k, i, pl.ds(j * 16, 16)])

    def adds(c, b):
        wait_x(c, b)
        adds_e01(c, b)
        wait_g(c, b)
        adds_rows(c, b)

    start_x(0, 0)
    start_g(0, 0)
    start_x(1, 1)
    start_g(1, 1)

    def step(c, b, pb, prefetch, first=False):
        adds(c, b)
        start_out(c, b)
        if prefetch:
            def pf():
                if not first:
                    wait_out(c - 1, pb)
                start_g(c + 2, pb)
                start_x(c + 2, pb)
            if first:
                pf()
            else:
                pl.when(c + 2 < NCHUNK)(pf)

    step(0, 0, 2, True, first=True)

    def triple(q, carry):
        for k in (1, 2, 3):
            c = 3 * q + k
            step(c, k % 3, (k + 2) % 3, True)
        return carry

    lax.fori_loop(0, (NCHUNK - 2) // 3, triple, 0)

    cL = NCHUNK - 1
    step(cL, cL % 3, 0, False)
    wait_out(cL - 2, (cL - 2) % 3)
    wait_out(cL - 1, (cL - 1) % 3)
    wait_out(cL, cL % 3)


@jax.jit
def _run(x2d, t1d, E0, E1, E2, E3):
    mesh = plsc.VectorSubcoreMesh(core_axis_name="c", subcore_axis_name="s")
    launch = functools.partial(
        pl.kernel,
        out_type=jax.ShapeDtypeStruct((TOK, D), jnp.float32),
        mesh=mesh,
        scratch_types=[
            pltpu.VMEM((3, T, D), jnp.float32),
            pltpu.VMEM((3, 2, T, ED), jnp.float32),
            pltpu.VMEM(((PERIODS[0] + PERIODS[1]) * ED,), jnp.float32),
            pltpu.VMEM((TPW + 16,), jnp.int32),
            pltpu.VMEM((2, TPW), jnp.int32),
            pltpu.SemaphoreType.DMA((3,)),
            pltpu.SemaphoreType.DMA((3,)),
            pltpu.SemaphoreType.DMA((3,)),
        ],
    )(_body)
    return launch(x2d, t1d, E0.reshape(-1), E1.reshape(-1), E2, E3)


def kernel(x, time_indices, E0, E1, E2, E3):
    B, S, _ = x.shape
    out = _run(
        x.reshape(TOK, D),
        time_indices.reshape(TOK).astype(jnp.int32),
        E0, E1, E2, E3,
    )
    return out.reshape(B, S, D)
